# baseline (speedup 1.0000x reference)
import functools

import jax
import jax.numpy as jnp
import numpy as np
from jax import lax
from jax.experimental import pallas as pl
from jax.experimental.pallas import tpu as pltpu
from jax.experimental.pallas import tpu_sc as plsc

B = 256
G = 8192

_RAND_INDEX = [
    5316, 4114, 1207, 7361, 653, 7531, 2433, 2343, 6150, 5378, 552, 6130,
    7577, 475, 8140, 1810, 5707, 4994, 2883, 519, 3638, 651, 2316, 7875,
    3180, 1553, 7152, 539, 6428, 3383, 6405, 676, 1493, 2094, 3123, 2068,
    4910, 6066, 3921, 6125, 5895, 5700, 3735, 381, 7033, 4288, 3388, 6820,
    4899, 5645, 5780, 7899, 978, 371, 2040, 439, 2059, 5458, 1883, 3001,
    6937, 7942, 1824, 3720, 1667, 6521, 4404, 4704, 578, 3257, 3244, 3909,
    7155, 6189, 3748, 508, 3374, 5034, 1585, 5217, 5655, 1744, 7605, 5932,
    7601, 1848, 4952, 2104, 5544, 7166, 5215, 3733, 2878, 6840, 1171, 7129,
    3063, 1503, 7354, 5089, 4913, 3420, 4389, 5668, 5247, 6604, 2622, 4642,
    4761, 3225, 6011, 5119, 1315, 7868, 3393, 3683, 7557, 6408, 3934, 5731,
    1667, 832, 3330, 1091, 5474, 3284, 3278, 4998, 593, 1628, 3351, 2722,
    2722, 93, 2287, 6886, 857, 874, 2257, 2881, 6343, 7756, 7835, 3932,
    6546, 7590, 7467, 5891, 625, 2593, 3015, 1586, 1643, 6448, 8096, 1619,
    3758, 1169, 7563, 1434, 5166, 1620, 5910, 2643, 3726, 1755, 1359, 4639,
    408, 7471, 3879, 4468, 2788, 7393, 1302, 2575, 1902, 456, 2082, 7219,
    6916, 5221, 2806, 3591, 6527, 7302, 7621, 6264, 3756, 2197, 6509, 4269,
    5812, 1421, 7255, 6611, 3904, 6603, 1718, 5566, 5609, 2089, 7601, 5989,
    1247, 2352, 7874, 354, 6639, 609, 6761, 591, 2073, 6068, 6231, 7859,
    5117, 4210, 4043, 249, 6532, 6092, 497, 2219, 4336, 7533, 1568, 788,
    5468, 5684, 1149, 2249, 2463, 1753, 438, 3755, 2280, 6100, 3767, 7525,
    1525, 5568, 1568, 5114, 6289, 889, 7445, 2520, 3648, 5802, 7074, 225,
    7932, 1511, 5370, 2832,
]
_AOFF = np.asarray(_RAND_INDEX, dtype=np.int32) * 3
K = int(0.6 * G)
NC, NS, L = 2, 16, 16
NW = NC * NS
ROWS = B // NW
NCHUNK = G // L
NBINS = 256
HIST = L * NBINS

_mesh = plsc.VectorSubcoreMesh(
    core_axis_name="c", subcore_axis_name="s", num_cores=NC, num_subcores=NS
)


@functools.partial(
    pl.kernel,
    out_type=jax.ShapeDtypeStruct((B, G), jnp.int32),
    mesh=_mesh,
    scratch_types=[
        pltpu.VMEM((2 * G * 3,), jnp.float32),
        pltpu.VMEM((G,), jnp.int32),
        pltpu.VMEM((2 * G,), jnp.int32),
        pltpu.VMEM((HIST,), jnp.int32),
        pltpu.VMEM((NBINS,), jnp.int32),
        pltpu.SMEM((B,), jnp.int32),
        pltpu.SemaphoreType.DMA,
        pltpu.SemaphoreType.DMA,
        pltpu.SemaphoreType.DMA,
    ],
    compiler_params=pltpu.CompilerParams(
        use_tc_tiling_on_sc=False, needs_layout_passes=False
    ),
)
def _mask_kernel(
    center_hbm, out_hbm, pts, bits, mask, hist, hsum, ivv, isem, osem0, osem1
):
    wid = lax.axis_index("s") * NC + lax.axis_index("c")
    zeros = jnp.zeros((L,), jnp.int32)
    ones = jnp.ones((L,), jnp.int32)
    lane = lax.iota(jnp.int32, L)
    lane_off = lane * NBINS
    stride3 = lane * 3

    for j in range(B):
        ivv[j] = jnp.int32(int(_AOFF[j]))

    @plsc.parallel_loop(0, HIST // L, unroll=8)
    def _clr(i):
        hist[pl.ds(i * L, L)] = zeros

    def scan_hist(r0):
        @plsc.parallel_loop(0, NBINS // L, unroll=2)
        def _sum(j):
            s = zeros
            for l in range(L):
                off = j * L
                v = hist[pl.ds(off + l * NBINS, L)]
                hist[pl.ds(off + l * NBINS, L)] = zeros
                s = s + v
            hsum[pl.ds(j * L, L)] = s

        gsums = [jnp.sum(hsum[pl.ds(j * L, L)]) for j in range(NBINS // L)]
        tot = 0
        jstar = 0
        cbase = 0
        for g in gsums:
            le_s = (tot + g) <= r0
            jstar = jstar + jnp.where(le_s, 1, 0)
            cbase = cbase + jnp.where(le_s, g, 0)
            tot = tot + g
        v = hsum[pl.ds(jstar * L, L)]
        cum = plsc.cumsum(v) + cbase
        le = cum <= r0
        cross = jnp.logical_and(cum - v <= r0, jnp.logical_not(le))
        b = jstar * L + jnp.sum(jnp.where(le, 1, 0))
        below = jnp.maximum(jnp.max(jnp.where(le, cum, 0)), cbase)
        hb = jnp.max(jnp.where(cross, v, 0))
        return b, r0 - below, hb

    pltpu.sync_copy(center_hbm.at[wid * ROWS], pts.at[pl.ds(0, G * 3)])

    def row_body(i, c):
        row = wid * ROWS + i
        par = jnp.bitwise_and(i, 1)
        base = par * (G * 3)
        nbase = (1 - par) * (G * 3)
        obase = par * G

        @pl.when(i < ROWS - 1)
        def _prefetch():
            pltpu.async_copy(
                center_hbm.at[row + 1], pts.at[pl.ds(nbase, G * 3)], isem
            )

        iv = jnp.full((L,), ivv[row] + base, jnp.int32)
        ax = plsc.load_gather(pts, [iv])
        ay = plsc.load_gather(pts, [iv + 1])
        az = plsc.load_gather(pts, [iv + 2])

        @plsc.parallel_loop(0, NCHUNK, unroll=8)
        def pass_a(cc):
            i0 = base + cc * (3 * L) + stride3
            x = plsc.load_gather(pts, [i0])
            y = plsc.load_gather(pts, [i0 + 1])
            z = plsc.load_gather(pts, [i0 + 2])
            dx = x - ax
            dy = y - ay
            dz = z - az
            d2 = dx * dx + dy * dy + dz * dz
            bv = plsc.bitcast(d2, jnp.int32)
            bits[pl.ds(cc * L, L)] = bv
            bkt = lax.shift_right_logical(bv, 23)
            plsc.addupdate_scatter(hist, [lane_off + bkt], ones)

        b1, r0, _ = scan_hist(K - 1)

        def run_pass(hi_shift, lo_shift, width_mask, prefix):
            @plsc.parallel_loop(0, NCHUNK, unroll=8)
            def _p(cc):
                bv = bits[pl.ds(cc * L, L)]
                m = lax.shift_right_logical(bv, hi_shift) == prefix
                bkt = jnp.bitwise_and(
                    lax.shift_right_logical(bv, lo_shift), width_mask
                )
                plsc.addupdate_scatter(hist, [lane_off + bkt], ones, mask=m)

        run_pass(23, 15, 0xFF, b1)
        b2, r0, _ = scan_hist(r0)
        p2 = b1 * 256 + b2
        run_pass(15, 7, 0xFF, p2)
        b3, r0, _ = scan_hist(r0)
        p3 = p2 * 256 + b3
        run_pass(7, 0, 0x7F, p3)
        b4, r0, hb = scan_hist(r0)
        t = p3 * 128 + b4
        need = r0 + 1

        @pl.when(jnp.logical_and(i >= 2, par == 0))
        def _drain0():
            pltpu.make_async_copy(
                mask.at[pl.ds(0, G)], out_hbm.at[row], osem0
            ).wait()

        @pl.when(jnp.logical_and(i >= 2, par == 1))
        def _drain1():
            pltpu.make_async_copy(
                mask.at[pl.ds(G, G)], out_hbm.at[row], osem1
            ).wait()

        def pass_e_fast(_):
            @plsc.parallel_loop(0, NCHUNK, unroll=8)
            def body(cc):
                bv = bits[pl.ds(cc * L, L)]
                mask[pl.ds(obase + cc * L, L)] = jnp.where(bv <= t, 1, 0)

            return 0

        def pass_e_tie(_):
            def body(cc, cnt):
                bv = bits[pl.ds(cc * L, L)]
                lt = bv < t
                eq = bv == t
                eqi = jnp.where(eq, 1, 0)
                pc = plsc.cumsum(eqi) + cnt
                sel = jnp.logical_and(eq, pc <= need)
                mask[pl.ds(obase + cc * L, L)] = jnp.where(
                    jnp.logical_or(lt, sel), 1, 0
                )
                return cnt + jnp.sum(eqi)

            return lax.fori_loop(0, NCHUNK, body, 0, unroll=8)

        lax.cond(hb == need, pass_e_fast, pass_e_tie, 0)

        @pl.when(par == 0)
        def _out0():
            pltpu.async_copy(mask.at[pl.ds(0, G)], out_hbm.at[row], osem0)

        @pl.when(par == 1)
        def _out1():
            pltpu.async_copy(mask.at[pl.ds(G, G)], out_hbm.at[row], osem1)

        @pl.when(i < ROWS - 1)
        def _wait_in():
            pltpu.make_async_copy(
                center_hbm.at[row + 1], pts.at[pl.ds(nbase, G * 3)], isem
            ).wait()

        return c

    lax.fori_loop(0, ROWS, row_body, 0)

    pltpu.make_async_copy(
        mask.at[pl.ds(0, G)], out_hbm.at[wid * ROWS + ROWS - 2], osem0
    ).wait()
    pltpu.make_async_copy(
        mask.at[pl.ds(G, G)], out_hbm.at[wid * ROWS + ROWS - 1], osem1
    ).wait()


def kernel(center):
    b, g, _ = center.shape
    flat = center.reshape(b, g * 3)
    out = _mask_kernel(flat)
    return out.astype(jnp.bool_)


# speedup vs baseline: 1.0093x; 1.0093x over previous
import functools

import jax
import jax.numpy as jnp
import numpy as np
from jax import lax
from jax.experimental import pallas as pl
from jax.experimental.pallas import tpu as pltpu
from jax.experimental.pallas import tpu_sc as plsc

B = 256
G = 8192

_RAND_INDEX = [
    5316, 4114, 1207, 7361, 653, 7531, 2433, 2343, 6150, 5378, 552, 6130,
    7577, 475, 8140, 1810, 5707, 4994, 2883, 519, 3638, 651, 2316, 7875,
    3180, 1553, 7152, 539, 6428, 3383, 6405, 676, 1493, 2094, 3123, 2068,
    4910, 6066, 3921, 6125, 5895, 5700, 3735, 381, 7033, 4288, 3388, 6820,
    4899, 5645, 5780, 7899, 978, 371, 2040, 439, 2059, 5458, 1883, 3001,
    6937, 7942, 1824, 3720, 1667, 6521, 4404, 4704, 578, 3257, 3244, 3909,
    7155, 6189, 3748, 508, 3374, 5034, 1585, 5217, 5655, 1744, 7605, 5932,
    7601, 1848, 4952, 2104, 5544, 7166, 5215, 3733, 2878, 6840, 1171, 7129,
    3063, 1503, 7354, 5089, 4913, 3420, 4389, 5668, 5247, 6604, 2622, 4642,
    4761, 3225, 6011, 5119, 1315, 7868, 3393, 3683, 7557, 6408, 3934, 5731,
    1667, 832, 3330, 1091, 5474, 3284, 3278, 4998, 593, 1628, 3351, 2722,
    2722, 93, 2287, 6886, 857, 874, 2257, 2881, 6343, 7756, 7835, 3932,
    6546, 7590, 7467, 5891, 625, 2593, 3015, 1586, 1643, 6448, 8096, 1619,
    3758, 1169, 7563, 1434, 5166, 1620, 5910, 2643, 3726, 1755, 1359, 4639,
    408, 7471, 3879, 4468, 2788, 7393, 1302, 2575, 1902, 456, 2082, 7219,
    6916, 5221, 2806, 3591, 6527, 7302, 7621, 6264, 3756, 2197, 6509, 4269,
    5812, 1421, 7255, 6611, 3904, 6603, 1718, 5566, 5609, 2089, 7601, 5989,
    1247, 2352, 7874, 354, 6639, 609, 6761, 591, 2073, 6068, 6231, 7859,
    5117, 4210, 4043, 249, 6532, 6092, 497, 2219, 4336, 7533, 1568, 788,
    5468, 5684, 1149, 2249, 2463, 1753, 438, 3755, 2280, 6100, 3767, 7525,
    1525, 5568, 1568, 5114, 6289, 889, 7445, 2520, 3648, 5802, 7074, 225,
    7932, 1511, 5370, 2832,
]
_AOFF = np.asarray(_RAND_INDEX, dtype=np.int32) * 3
K = int(0.6 * G)
NC, NS, L = 2, 16, 16
NW = NC * NS
ROWS = B // NW
NCHUNK = G // L
NBINS = 256
HIST = L * NBINS

_mesh = plsc.VectorSubcoreMesh(
    core_axis_name="c", subcore_axis_name="s", num_cores=NC, num_subcores=NS
)


@functools.partial(
    pl.kernel,
    out_type=jax.ShapeDtypeStruct((B, G), jnp.int32),
    mesh=_mesh,
    scratch_types=[
        pltpu.VMEM((2 * G * 3,), jnp.float32),
        pltpu.VMEM((G,), jnp.int32),
        pltpu.VMEM((2 * G,), jnp.int32),
        pltpu.VMEM((HIST,), jnp.int32),
        pltpu.SMEM((B,), jnp.int32),
        pltpu.SemaphoreType.DMA,
        pltpu.SemaphoreType.DMA,
        pltpu.SemaphoreType.DMA,
    ],
    compiler_params=pltpu.CompilerParams(
        use_tc_tiling_on_sc=False, needs_layout_passes=False
    ),
)
def _mask_kernel(
    center_hbm, out_hbm, pts, bits, mask, hist, ivv, isem, osem0, osem1
):
    wid = lax.axis_index("s") * NC + lax.axis_index("c")
    zeros = jnp.zeros((L,), jnp.int32)
    ones = jnp.ones((L,), jnp.int32)
    lane = lax.iota(jnp.int32, L)
    lane_off = lane * NBINS
    stride3 = lane * 3

    for j in range(B):
        ivv[j] = jnp.int32(int(_AOFF[j]))

    @plsc.parallel_loop(0, HIST // L, unroll=8)
    def _clr(i):
        hist[pl.ds(i * L, L)] = zeros

    def scan_hist(r0):
        def grp(j, carry):
            tot, b_acc, below, hb = carry
            s = zeros
            for l in range(L):
                off = j * L
                v = hist[pl.ds(off + l * NBINS, L)]
                hist[pl.ds(off + l * NBINS, L)] = zeros
                s = s + v
            cum = plsc.cumsum(s) + tot
            le = cum <= r0
            cross = jnp.logical_and(cum - s <= r0, jnp.logical_not(le))
            b_acc = b_acc + jnp.sum(jnp.where(le, 1, 0))
            below = below + jnp.sum(jnp.where(le, s, 0))
            hb = hb + jnp.sum(jnp.where(cross, s, 0))
            tot = tot + jnp.sum(s)
            return (tot, b_acc, below, hb)

        _, b, below, hb = lax.fori_loop(0, NBINS // L, grp, (0, 0, 0, 0))
        return b, r0 - below, hb

    pltpu.sync_copy(center_hbm.at[wid * ROWS], pts.at[pl.ds(0, G * 3)])

    def row_body(i, c):
        row = wid * ROWS + i
        par = jnp.bitwise_and(i, 1)
        base = par * (G * 3)
        nbase = (1 - par) * (G * 3)
        obase = par * G

        @pl.when(i < ROWS - 1)
        def _prefetch():
            pltpu.async_copy(
                center_hbm.at[row + 1], pts.at[pl.ds(nbase, G * 3)], isem
            )

        iv = jnp.full((L,), ivv[row] + base, jnp.int32)
        ax = plsc.load_gather(pts, [iv])
        ay = plsc.load_gather(pts, [iv + 1])
        az = plsc.load_gather(pts, [iv + 2])

        @plsc.parallel_loop(0, NCHUNK, unroll=8)
        def pass_a(cc):
            i0 = base + cc * (3 * L) + stride3
            x = plsc.load_gather(pts, [i0])
            y = plsc.load_gather(pts, [i0 + 1])
            z = plsc.load_gather(pts, [i0 + 2])
            dx = x - ax
            dy = y - ay
            dz = z - az
            d2 = dx * dx + dy * dy + dz * dz
            bv = plsc.bitcast(d2, jnp.int32)
            bits[pl.ds(cc * L, L)] = bv
            bkt = lax.shift_right_logical(bv, 23)
            plsc.addupdate_scatter(hist, [lane_off + bkt], ones)

        b1, r0, _ = scan_hist(K - 1)

        def run_pass(hi_shift, lo_shift, width_mask, prefix):
            @plsc.parallel_loop(0, NCHUNK, unroll=8)
            def _p(cc):
                bv = bits[pl.ds(cc * L, L)]
                m = lax.shift_right_logical(bv, hi_shift) == prefix
                bkt = jnp.bitwise_and(
                    lax.shift_right_logical(bv, lo_shift), width_mask
                )
                plsc.addupdate_scatter(hist, [lane_off + bkt], ones, mask=m)

        run_pass(23, 15, 0xFF, b1)
        b2, r0, _ = scan_hist(r0)
        p2 = b1 * 256 + b2
        run_pass(15, 7, 0xFF, p2)
        b3, r0, _ = scan_hist(r0)
        p3 = p2 * 256 + b3
        run_pass(7, 0, 0x7F, p3)
        b4, r0, hb = scan_hist(r0)
        t = p3 * 128 + b4
        need = r0 + 1

        @pl.when(jnp.logical_and(i >= 2, par == 0))
        def _drain0():
            pltpu.make_async_copy(
                mask.at[pl.ds(0, G)], out_hbm.at[row], osem0
            ).wait()

        @pl.when(jnp.logical_and(i >= 2, par == 1))
        def _drain1():
            pltpu.make_async_copy(
                mask.at[pl.ds(G, G)], out_hbm.at[row], osem1
            ).wait()

        def pass_e_fast(_):
            @plsc.parallel_loop(0, NCHUNK, unroll=8)
            def body(cc):
                bv = bits[pl.ds(cc * L, L)]
                mask[pl.ds(obase + cc * L, L)] = jnp.where(bv <= t, 1, 0)

            return 0

        def pass_e_tie(_):
            def body(cc, cnt):
                bv = bits[pl.ds(cc * L, L)]
                lt = bv < t
                eq = bv == t
                eqi = jnp.where(eq, 1, 0)
                pc = plsc.cumsum(eqi) + cnt
                sel = jnp.logical_and(eq, pc <= need)
                mask[pl.ds(obase + cc * L, L)] = jnp.where(
                    jnp.logical_or(lt, sel), 1, 0
                )
                return cnt + jnp.sum(eqi)

            return lax.fori_loop(0, NCHUNK, body, 0, unroll=8)

        lax.cond(hb == need, pass_e_fast, pass_e_tie, 0)

        @pl.when(par == 0)
        def _out0():
            pltpu.async_copy(mask.at[pl.ds(0, G)], out_hbm.at[row], osem0)

        @pl.when(par == 1)
        def _out1():
            pltpu.async_copy(mask.at[pl.ds(G, G)], out_hbm.at[row], osem1)

        @pl.when(i < ROWS - 1)
        def _wait_in():
            pltpu.make_async_copy(
                center_hbm.at[row + 1], pts.at[pl.ds(nbase, G * 3)], isem
            ).wait()

        return c

    lax.fori_loop(0, ROWS, row_body, 0)

    pltpu.make_async_copy(
        mask.at[pl.ds(0, G)], out_hbm.at[wid * ROWS + ROWS - 2], osem0
    ).wait()
    pltpu.make_async_copy(
        mask.at[pl.ds(G, G)], out_hbm.at[wid * ROWS + ROWS - 1], osem1
    ).wait()


def kernel(center):
    b, g, _ = center.shape
    flat = center.reshape(b, g * 3)
    out = _mask_kernel(flat)
    return out.astype(jnp.bool_)
